# Initial kernel scaffold; baseline (speedup 1.0000x reference)
import jax
import jax.numpy as jnp
from jax.experimental import pallas as pl


def kernel(chosen, rejected, E, W, b):
    raise NotImplementedError("write your pallas kernel here")


# speedup vs baseline: 200.2611x; 200.2611x over previous
import functools

import jax
import jax.numpy as jnp
from jax import lax
from jax.experimental import pallas as pl
from jax.experimental.pallas import tpu as pltpu
from jax.experimental.pallas import tpu_sc as plsc

ROWS, COLS = 16384, 200
N = ROWS * COLS
NC, NS, LANES = 2, 16, 16
NW = NC * NS
PER_W = N // NW
CHUNK = 12800
NCH = PER_W // CHUNK
VPC = CHUNK // LANES


def _make_sc_call():
    mesh = plsc.VectorSubcoreMesh(core_axis_name="c", subcore_axis_name="s")

    @functools.partial(
        pl.kernel,
        mesh=mesh,
        out_type=jax.ShapeDtypeStruct((NW, LANES), jnp.float32),
        compiler_params=pltpu.CompilerParams(needs_layout_passes=False),
        scratch_types=[
            pltpu.VMEM((CHUNK,), jnp.int32),
            pltpu.VMEM((CHUNK,), jnp.int32),
            pltpu.VMEM((4, LANES), jnp.float32),
            pltpu.VMEM((LANES,), jnp.float32),
            pltpu.VMEM((LANES,), jnp.float32),
        ],
    )
    def sc_call(ch_hbm, rj_hbm, ew_hbm, out_hbm, a_v, b_v, ew_v, stab, accv):
        cid = lax.axis_index("c")
        sid = lax.axis_index("s")
        wid = sid * NC + cid
        base = wid * PER_W

        pltpu.sync_copy(ew_hbm, ew_v)
        et0 = ew_v[0, :]
        et1 = ew_v[1, :]
        et2 = ew_v[2, :]
        wrow = ew_v[3, :]
        w0 = wrow[0]
        w1 = wrow[1]
        w2 = wrow[2]
        b0 = wrow[3]
        h = et0 * w0 + et1 * w1 + et2 * w2 + b0
        stab[...] = 1.0 / (1.0 + jnp.exp(-h))

        nine = jnp.full((LANES,), 9, jnp.int32)

        def chunk_body(j, acc):
            off = base + j * CHUNK
            pltpu.sync_copy(ch_hbm.at[pl.ds(off, CHUNK)], a_v)
            pltpu.sync_copy(rj_hbm.at[pl.ds(off, CHUNK)], b_v)

            def vec_body(i, acc):
                av = jnp.minimum(a_v[pl.ds(i * LANES, LANES)], nine)
                bv = jnp.minimum(b_v[pl.ds(i * LANES, LANES)], nine)
                sa = plsc.load_gather(stab, [av])
                sb = plsc.load_gather(stab, [bv])
                return acc + (sa - sb)

            return lax.fori_loop(0, VPC, vec_body, acc)

        acc = lax.fori_loop(0, NCH, chunk_body, jnp.zeros((LANES,), jnp.float32))
        accv[...] = acc
        pltpu.sync_copy(accv, out_hbm.at[wid])

    return sc_call


_SC_CALL = _make_sc_call()


def kernel(chosen, rejected, E, W, b):
    ch = chosen.reshape(-1)
    rj = rejected.reshape(-1)
    ew = (
        jnp.zeros((4, LANES), jnp.float32)
        .at[0:3, 0:10].set(E.T)
        .at[3, 0:3].set(W[0])
        .at[3, 3].set(b[0])
    )
    partials = _SC_CALL(ch, rj, ew)
    diff = jnp.sum(partials) / jnp.float32(N)
    loss = diff * diff
    return (loss, diff)


# speedup vs baseline: 245.9864x; 1.2283x over previous
import functools

import jax
import jax.numpy as jnp
from jax import lax
from jax.experimental import pallas as pl
from jax.experimental.pallas import tpu as pltpu
from jax.experimental.pallas import tpu_sc as plsc

ROWS, COLS = 16384, 200
N = ROWS * COLS
NC, NS, LANES = 2, 16, 16
NW = NC * NS
PER_W = N // NW
CHUNK = 25600
NCH = PER_W // CHUNK
PAIRS = CHUNK // LANES


def _make_sc_call():
    mesh = plsc.VectorSubcoreMesh(core_axis_name="c", subcore_axis_name="s")

    @functools.partial(
        pl.kernel,
        mesh=mesh,
        out_type=jax.ShapeDtypeStruct((NW, LANES), jnp.float32),
        compiler_params=pltpu.CompilerParams(needs_layout_passes=False),
        scratch_types=[
            pltpu.VMEM((CHUNK,), jnp.int32),
            pltpu.VMEM((CHUNK,), jnp.int32),
            pltpu.VMEM((CHUNK,), jnp.int32),
            pltpu.VMEM((CHUNK,), jnp.int32),
            pltpu.VMEM((4, LANES), jnp.float32),
            pltpu.VMEM((LANES,), jnp.float32),
            pltpu.VMEM((8 * LANES,), jnp.float32),
            pltpu.VMEM((LANES,), jnp.float32),
            pltpu.SemaphoreType.DMA,
            pltpu.SemaphoreType.DMA,
            pltpu.SemaphoreType.DMA,
            pltpu.SemaphoreType.DMA,
        ],
    )
    def sc_call(ch_hbm, rj_hbm, ew_hbm, out_hbm,
                a0, a1, b0, b1, ew_v, stab, tab2, accv,
                sa0, sa1, sb0, sb1):
        cid = lax.axis_index("c")
        sid = lax.axis_index("s")
        wid = sid * NC + cid
        base = wid * PER_W

        a_bufs, b_bufs = (a0, a1), (b0, b1)
        a_sems, b_sems = (sa0, sa1), (sb0, sb1)

        def start(j):
            buf = j % 2
            off = base + j * CHUNK
            ha = pltpu.async_copy(ch_hbm.at[pl.ds(off, CHUNK)],
                                  a_bufs[buf], a_sems[buf])
            hb = pltpu.async_copy(rj_hbm.at[pl.ds(off, CHUNK)],
                                  b_bufs[buf], b_sems[buf])
            return ha, hb

        pending = start(0)

        pltpu.sync_copy(ew_hbm, ew_v)
        et0 = ew_v[0, :]
        et1 = ew_v[1, :]
        et2 = ew_v[2, :]
        wrow = ew_v[3, :]
        h = et0 * wrow[0] + et1 * wrow[1] + et2 * wrow[2] + wrow[3]
        stab[...] = 1.0 / (1.0 + jnp.exp(-h))
        iota = lax.iota(jnp.int32, LANES)
        for k in range(8):
            p = k * LANES + iota
            ia = jnp.minimum(p // 10, 15)
            ib = p - (p // 10) * 10
            ta = plsc.load_gather(stab, [ia])
            tb = plsc.load_gather(stab, [ib])
            tab2[pl.ds(k * LANES, LANES)] = ta - tb

        nine = jnp.full((LANES,), 9, jnp.int32)
        ten = jnp.full((LANES,), 10, jnp.int32)
        zero = jnp.zeros((LANES,), jnp.float32)
        accs = (zero, zero)

        for j in range(NCH):
            nxt = start(j + 1) if j + 1 < NCH else None
            pending[0].wait()
            pending[1].wait()
            a_cur, b_cur = a_bufs[j % 2], b_bufs[j % 2]

            def body(i, accs, a_cur=a_cur, b_cur=b_cur):
                o = i * (2 * LANES)
                av0 = jnp.minimum(a_cur[pl.ds(o, LANES)], nine)
                bv0 = jnp.minimum(b_cur[pl.ds(o, LANES)], nine)
                av1 = jnp.minimum(a_cur[pl.ds(o + LANES, LANES)], nine)
                bv1 = jnp.minimum(b_cur[pl.ds(o + LANES, LANES)], nine)
                t0 = plsc.load_gather(tab2, [av0 * ten + bv0])
                t1 = plsc.load_gather(tab2, [av1 * ten + bv1])
                return (accs[0] + t0, accs[1] + t1)

            accs = plsc.parallel_loop(
                0, PAIRS // 2, unroll=4, carry=accs)(body)
            pending = nxt

        accv[...] = accs[0] + accs[1]
        pltpu.sync_copy(accv, out_hbm.at[wid])

    return sc_call


_SC_CALL = _make_sc_call()


def kernel(chosen, rejected, E, W, b):
    ch = chosen.reshape(-1)
    rj = rejected.reshape(-1)
    ew = (
        jnp.zeros((4, LANES), jnp.float32)
        .at[0:3, 0:10].set(E.T)
        .at[3, 0:3].set(W[0])
        .at[3, 3].set(b[0])
    )
    partials = _SC_CALL(ch, rj, ew)
    diff = jnp.sum(partials) / jnp.float32(N)
    loss = diff * diff
    return (loss, diff)


# speedup vs baseline: 365.1227x; 1.4843x over previous
import functools

import jax
import jax.numpy as jnp
from jax import lax
from jax.experimental import pallas as pl
from jax.experimental.pallas import tpu as pltpu
from jax.experimental.pallas import tpu_sc as plsc

ROWS, COLS = 16384, 200
N = ROWS * COLS
NC, NS, LANES = 2, 16, 16
NW = NC * NS
ROWS_W = ROWS // NW
RCHUNK = 64
NCH = ROWS_W // RCHUNK
FULL = COLS // LANES
TAIL = COLS - FULL * LANES


def _make_sc_call():
    mesh = plsc.VectorSubcoreMesh(core_axis_name="c", subcore_axis_name="s")

    @functools.partial(
        pl.kernel,
        mesh=mesh,
        out_type=jax.ShapeDtypeStruct((NW, LANES), jnp.float32),
        compiler_params=pltpu.CompilerParams(
            needs_layout_passes=False, use_tc_tiling_on_sc=True),
        scratch_types=[
            pltpu.VMEM((RCHUNK, COLS), jnp.int32),
            pltpu.VMEM((RCHUNK, COLS), jnp.int32),
            pltpu.VMEM((RCHUNK, COLS), jnp.int32),
            pltpu.VMEM((RCHUNK, COLS), jnp.int32),
            pltpu.VMEM((4, LANES), jnp.float32),
            pltpu.VMEM((LANES,), jnp.float32),
            pltpu.VMEM((8 * LANES,), jnp.float32),
            pltpu.VMEM((LANES,), jnp.float32),
            pltpu.SemaphoreType.DMA,
            pltpu.SemaphoreType.DMA,
            pltpu.SemaphoreType.DMA,
            pltpu.SemaphoreType.DMA,
        ],
    )
    def sc_call(ch_hbm, rj_hbm, ew_hbm, out_hbm,
                a0, a1, b0, b1, ew_v, stab, tab2, accv,
                sa0, sa1, sb0, sb1):
        cid = lax.axis_index("c")
        sid = lax.axis_index("s")
        wid = sid * NC + cid
        row0 = wid * ROWS_W

        a_bufs, b_bufs = (a0, a1), (b0, b1)
        a_sems, b_sems = (sa0, sa1), (sb0, sb1)

        def start(j):
            buf = j % 2
            r = row0 + j * RCHUNK
            ha = pltpu.async_copy(ch_hbm.at[pl.ds(r, RCHUNK)],
                                  a_bufs[buf], a_sems[buf])
            hb = pltpu.async_copy(rj_hbm.at[pl.ds(r, RCHUNK)],
                                  b_bufs[buf], b_sems[buf])
            return ha, hb

        pending = start(0)

        pltpu.sync_copy(ew_hbm, ew_v)
        et0 = ew_v[0, :]
        et1 = ew_v[1, :]
        et2 = ew_v[2, :]
        wrow = ew_v[3, :]
        h = et0 * wrow[0] + et1 * wrow[1] + et2 * wrow[2] + wrow[3]
        stab[...] = 1.0 / (1.0 + jnp.exp(-h))
        iota = lax.iota(jnp.int32, LANES)
        for k in range(8):
            p = k * LANES + iota
            ia = jnp.minimum(p // 10, 15)
            ib = p - (p // 10) * 10
            ta = plsc.load_gather(stab, [ia])
            tb = plsc.load_gather(stab, [ib])
            tab2[pl.ds(k * LANES, LANES)] = ta - tb

        ten = jnp.full((LANES,), 10, jnp.int32)
        hi8 = iota >= TAIL
        zero = jnp.zeros((LANES,), jnp.float32)
        accs = (zero, zero)

        for j in range(NCH):
            nxt = start(j + 1) if j + 1 < NCH else None
            pending[0].wait()
            pending[1].wait()
            a_cur, b_cur = a_bufs[j % 2], b_bufs[j % 2]

            def body(r, accs, a_cur=a_cur, b_cur=b_cur):
                acc0, acc1 = accs
                for k in range(FULL):
                    av = a_cur[r, pl.ds(k * LANES, LANES)]
                    bv = b_cur[r, pl.ds(k * LANES, LANES)]
                    t = plsc.load_gather(tab2, [av * ten + bv])
                    if k % 2 == 0:
                        acc0 = acc0 + t
                    else:
                        acc1 = acc1 + t
                av = a_cur[r, pl.ds(COLS - LANES, LANES)]
                bv = b_cur[r, pl.ds(COLS - LANES, LANES)]
                t = plsc.load_gather(tab2, [av * ten + bv])
                acc0 = acc0 + jnp.where(hi8, t, 0.0)
                return (acc0, acc1)

            accs = plsc.parallel_loop(0, RCHUNK, unroll=2, carry=accs)(body)
            pending = nxt

        accv[...] = accs[0] + accs[1]
        pltpu.sync_copy(accv, out_hbm.at[wid])

    return sc_call


_SC_CALL = _make_sc_call()


def kernel(chosen, rejected, E, W, b):
    ew = (
        jnp.zeros((4, LANES), jnp.float32)
        .at[0:3, 0:10].set(E.T)
        .at[3, 0:3].set(W[0])
        .at[3, 3].set(b[0])
    )
    partials = _SC_CALL(chosen, rejected, ew)
    diff = jnp.sum(partials) / jnp.float32(N)
    loss = diff * diff
    return (loss, diff)


# speedup vs baseline: 585.3884x; 1.6033x over previous
import functools

import jax
import jax.numpy as jnp
from jax import lax
from jax.experimental import pallas as pl
from jax.experimental.pallas import tpu as pltpu
from jax.experimental.pallas import tpu_sc as plsc

ROWS, COLS = 16384, 200
N = ROWS * COLS
NC, NS, LANES = 2, 16, 16
NW = NC * NS
TR, TCOL = COLS, ROWS
CW = TCOL // NW
RCH = 8
NCHK = TR // RCH
NBUF = 5
VPR = CW // LANES
GRP = 4
ITERS = RCH * VPR // GRP


def _make_sc_call():
    mesh = plsc.VectorSubcoreMesh(core_axis_name="c", subcore_axis_name="s")

    @functools.partial(
        pl.kernel,
        mesh=mesh,
        out_type=jax.ShapeDtypeStruct((NW, LANES), jnp.float32),
        compiler_params=pltpu.CompilerParams(
            needs_layout_passes=False, use_tc_tiling_on_sc=True),
        scratch_types=(
            [pltpu.VMEM((RCH, CW), jnp.int32) for _ in range(2 * NBUF)]
            + [
                pltpu.VMEM((4, LANES), jnp.float32),
                pltpu.VMEM((10 * LANES,), jnp.float32),
                pltpu.VMEM((LANES,), jnp.float32),
            ]
            + [pltpu.SemaphoreType.DMA for _ in range(2 * NBUF)]
        ),
    )
    def sc_call(ch_hbm, rj_hbm, ew_hbm, out_hbm, *scratch):
        a_bufs = scratch[0:NBUF]
        b_bufs = scratch[NBUF:2 * NBUF]
        ew_v = scratch[2 * NBUF]
        tab2 = scratch[2 * NBUF + 1]
        accv = scratch[2 * NBUF + 2]
        a_sems = scratch[2 * NBUF + 3:2 * NBUF + 3 + NBUF]
        b_sems = scratch[2 * NBUF + 3 + NBUF:2 * NBUF + 3 + 2 * NBUF]

        cid = lax.axis_index("c")
        sid = lax.axis_index("s")
        wid = sid * NC + cid
        col0 = wid * CW

        def issue(c):
            k = c % NBUF
            rows = pl.ds(c * RCH, RCH)
            cols = pl.ds(col0, CW)
            ha = pltpu.async_copy(ch_hbm.at[rows, cols], a_bufs[k], a_sems[k])
            hb = pltpu.async_copy(rj_hbm.at[rows, cols], b_bufs[k], b_sems[k])
            return ha, hb

        pending = [issue(c) for c in range(NBUF)]

        pltpu.sync_copy(ew_hbm, ew_v)
        et0 = ew_v[0, :]
        et1 = ew_v[1, :]
        et2 = ew_v[2, :]
        wrow = ew_v[3, :]
        h = et0 * wrow[0] + et1 * wrow[1] + et2 * wrow[2] + wrow[3]
        sv = 1.0 / (1.0 + jnp.exp(-h))
        for ka in range(10):
            tab2[pl.ds(ka * LANES, LANES)] = sv[ka] - sv

        zero = jnp.zeros((LANES,), jnp.float32)
        accs = (zero, zero, zero, zero)

        for c in range(NCHK):
            k = c % NBUF
            ha, hb = pending[k]
            ha.wait()
            hb.wait()
            a_cur, b_cur = a_bufs[k], b_bufs[k]

            def body(p, accs, a_cur=a_cur, b_cur=b_cur):
                r = p >> 3
                cb = (p & 7) * (GRP * LANES)
                new = []
                for t in range(GRP):
                    av = a_cur[r, pl.ds(cb + t * LANES, LANES)]
                    bv = b_cur[r, pl.ds(cb + t * LANES, LANES)]
                    idx = jax.lax.shift_left(av, 4) | bv
                    tv = plsc.load_gather(tab2, [idx])
                    new.append(accs[t] + tv)
                return tuple(new)

            accs = plsc.parallel_loop(0, ITERS, unroll=2, carry=accs)(body)
            if c + NBUF < NCHK:
                pending[k] = issue(c + NBUF)

        accv[...] = (accs[0] + accs[1]) + (accs[2] + accs[3])
        pltpu.sync_copy(accv, out_hbm.at[wid])

    return sc_call


_SC_CALL = _make_sc_call()


def kernel(chosen, rejected, E, W, b):
    ew = (
        jnp.zeros((4, LANES), jnp.float32)
        .at[0:3, 0:10].set(E.T)
        .at[3, 0:3].set(W[0])
        .at[3, 3].set(b[0])
    )
    partials = _SC_CALL(chosen.T, rejected.T, ew)
    diff = jnp.sum(partials) / jnp.float32(N)
    loss = diff * diff
    return (loss, diff)
